# baseline (speedup 1.0000x reference)
import functools

import jax
import jax.numpy as jnp
from jax import lax
from jax.experimental import pallas as pl
from jax.experimental.pallas import tpu as pltpu
from jax.experimental.pallas import tpu_sc as plsc

_NBUF = 2


def _pack_table(wT, V, D):
    CI = 2048
    NB = (V + CI - 1) // CI

    def body(x_ref, o_ref):
        x = x_ref[...]
        zs = []
        for q in range(4):
            parts = [
                x[:, 128 * (4 * q + k) : 128 * (4 * q + k) + 128].T
                for k in range(4)
            ]
            zs.append(jnp.concatenate(parts, axis=1))
        o_ref[...] = jnp.concatenate(zs, axis=0)

    return pl.pallas_call(
        body,
        grid=(NB,),
        in_specs=[pl.BlockSpec((D, CI), lambda i: (0, i))],
        out_specs=pl.BlockSpec((512, 4 * D), lambda i: (i, 0)),
        out_shape=jax.ShapeDtypeStruct((NB * 512, 4 * D), jnp.float32),
    )(wT)


def _gather_fn(B, D, VR, CH, num_ch, b_per_w, num_cores):
    mesh = plsc.VectorSubcoreMesh(core_axis_name="c", subcore_axis_name="s")

    @functools.partial(
        pl.kernel,
        mesh=mesh,
        out_type=jax.ShapeDtypeStruct((B, D), jnp.float32),
        scratch_types=[
            pltpu.VMEM((b_per_w,), jnp.int32),
            [pltpu.VMEM((CH, D), jnp.float32) for _ in range(_NBUF)],
            [pltpu.SemaphoreType.DMA for _ in range(_NBUF)],
            [pltpu.SemaphoreType.DMA for _ in range(_NBUF)],
        ],
        compiler_params=pltpu.CompilerParams(use_tc_tiling_on_sc=False),
    )
    def k(idx_hbm, table_hbm, out_hbm, idx_v, rows, g_sems, s_sems):
        wid = lax.axis_index("s") * num_cores + lax.axis_index("c")
        base = wid * b_per_w
        pltpu.sync_copy(idx_hbm.at[pl.ds(base, b_per_w)], idx_v)

        gathers = [None] * num_ch
        stores = [None] * num_ch

        def fire_gather(i):
            s = i % _NBUF
            gathers[i] = pltpu.async_copy(
                table_hbm.at[idx_v.at[pl.ds(i * CH, CH)]], rows[s], g_sems[s]
            )

        for b in range(min(_NBUF, num_ch)):
            fire_gather(b)
        for i in range(num_ch):
            s = i % _NBUF
            gathers[i].wait()
            stores[i] = pltpu.async_copy(
                rows[s], out_hbm.at[pl.ds(base + i * CH, CH)], s_sems[s]
            )
            nxt = i + _NBUF
            if nxt < num_ch:
                stores[i].wait()
                fire_gather(nxt)
        for i in range(max(0, num_ch - _NBUF), num_ch):
            stores[i].wait()

    return k


def _to_native_out(x7, Bt, F, D):
    BB = 128
    FD = F * D
    SL = (7 * 128) // BB if BB != 128 else 7

    def body(x_ref, o_ref):
        x = x_ref[...]
        y = x.reshape(BB, 7, 128)
        parts = [y[:, s, :].T for s in range(6)]
        parts.append(y[:, 6, :].T[0 : FD - 6 * 128, :])
        o_ref[...] = jnp.concatenate(parts, axis=0)

    return pl.pallas_call(
        body,
        grid=(Bt // BB,),
        in_specs=[pl.BlockSpec((BB * 7, 128), lambda i: (i, 0))],
        out_specs=pl.BlockSpec((FD, BB), lambda i: (0, i)),
        out_shape=jax.ShapeDtypeStruct((FD, Bt), jnp.float32),
    )(x7)


def kernel(indices, weights):
    Bt, F = indices.shape
    V, D = weights.shape
    FP = 28
    B2 = Bt * FP

    t4 = _pack_table(weights.T, V, D)
    VR = t4.shape[0] * 4
    table = t4.reshape(VR, D)

    idx = indices.astype(jnp.int32)
    idx = jnp.pad(idx, ((0, 0), (0, FP - F)))
    r = idx - (idx % 512) + 4 * (idx % 128) + (idx // 128) % 4
    idx_flat = r.reshape(B2)

    info = plsc.get_sparse_core_info()
    NW = info.num_cores * info.num_subcores
    b_per_w = B2 // NW
    CH = 896
    num_ch = b_per_w // CH
    assert b_per_w % CH == 0 and B2 % NW == 0

    flat = _gather_fn(B2, D, VR, CH, num_ch, b_per_w, info.num_cores)(
        idx_flat, table
    )

    phys = _to_native_out(flat.reshape(Bt * 7, 128), Bt, F, D)
    return jnp.transpose(phys.reshape(F, D, Bt), (2, 0, 1))


# speedup vs baseline: 1.6430x; 1.6430x over previous
import functools

import jax
import jax.numpy as jnp
from jax import lax
from jax.experimental import pallas as pl
from jax.experimental.pallas import tpu as pltpu
from jax.experimental.pallas import tpu_sc as plsc

_NBUF = 2


def _pack_table(wT, V, D):
    CI = 2048
    NB = (V + CI - 1) // CI

    def body(x_ref, o_ref):
        x = x_ref[...]
        zs = []
        for q in range(4):
            parts = [
                x[:, 128 * (4 * q + k) : 128 * (4 * q + k) + 128].T
                for k in range(4)
            ]
            zs.append(jnp.concatenate(parts, axis=1))
        o_ref[...] = jnp.concatenate(zs, axis=0)

    return pl.pallas_call(
        body,
        grid=(NB,),
        in_specs=[pl.BlockSpec((D, CI), lambda i: (0, i))],
        out_specs=pl.BlockSpec((512, 4 * D), lambda i: (i, 0)),
        out_shape=jax.ShapeDtypeStruct((NB * 512, 4 * D), jnp.float32),
    )(wT)


def _gather_fn(B, D, VR, CH, num_ch, b_per_w, num_cores):
    mesh = plsc.VectorSubcoreMesh(core_axis_name="c", subcore_axis_name="s")

    @functools.partial(
        pl.kernel,
        mesh=mesh,
        out_type=jax.ShapeDtypeStruct((B, D), jnp.float32),
        scratch_types=[
            pltpu.VMEM((b_per_w,), jnp.int32),
            [pltpu.VMEM((CH, D), jnp.float32) for _ in range(_NBUF)],
            [pltpu.SemaphoreType.DMA for _ in range(_NBUF)],
            [pltpu.SemaphoreType.DMA for _ in range(_NBUF)],
        ],
        compiler_params=pltpu.CompilerParams(use_tc_tiling_on_sc=False),
    )
    def k(idx_hbm, table_hbm, out_hbm, idx_v, rows, g_sems, s_sems):
        wid = lax.axis_index("s") * num_cores + lax.axis_index("c")
        base = wid * b_per_w
        pltpu.sync_copy(idx_hbm.at[pl.ds(base, b_per_w)], idx_v)

        gathers = [None] * num_ch
        stores = [None] * num_ch

        def fire_gather(i):
            s = i % _NBUF
            gathers[i] = pltpu.async_copy(
                table_hbm.at[idx_v.at[pl.ds(i * CH, CH)]], rows[s], g_sems[s]
            )

        for b in range(min(_NBUF, num_ch)):
            fire_gather(b)
        for i in range(num_ch):
            s = i % _NBUF
            gathers[i].wait()
            stores[i] = pltpu.async_copy(
                rows[s], out_hbm.at[pl.ds(base + i * CH, CH)], s_sems[s]
            )
            nxt = i + _NBUF
            if nxt < num_ch:
                stores[i].wait()
                fire_gather(nxt)
        for i in range(max(0, num_ch - _NBUF), num_ch):
            stores[i].wait()

    return k


def _to_native_out(x7, Bt, F, D):
    BB = 128
    FD = F * D
    SL = (7 * 128) // BB if BB != 128 else 7

    def body(x_ref, o_ref):
        x = x_ref[...]
        y = x.reshape(BB, 7, 128)
        parts = [y[:, s, :].T for s in range(6)]
        parts.append(y[:, 6, :].T[0 : FD - 6 * 128, :])
        o_ref[...] = jnp.concatenate(parts, axis=0)

    return pl.pallas_call(
        body,
        grid=(Bt // BB,),
        in_specs=[pl.BlockSpec((BB * 7, 128), lambda i: (i, 0))],
        out_specs=pl.BlockSpec((FD, BB), lambda i: (0, i)),
        out_shape=jax.ShapeDtypeStruct((FD, Bt), jnp.float32),
    )(x7)


def kernel(indices, weights):
    Bt, F = indices.shape
    V, D = weights.shape
    FP = 28
    B2 = Bt * FP

    t4 = _pack_table(weights.T, V, D)
    VR = t4.shape[0] * 4
    table = t4.reshape(VR, D)

    idx = indices.astype(jnp.int32)
    idx = jnp.pad(idx, ((0, 0), (0, FP - F)), mode="wrap")
    r = idx - (idx % 512) + 4 * (idx % 128) + (idx // 128) % 4
    idx_flat = r.reshape(B2)

    info = plsc.get_sparse_core_info()
    NW = info.num_cores * info.num_subcores
    b_per_w = B2 // NW
    CH = 896
    num_ch = b_per_w // CH
    assert b_per_w % CH == 0 and B2 % NW == 0

    flat = _gather_fn(B2, D, VR, CH, num_ch, b_per_w, info.num_cores)(
        idx_flat, table
    )

    phys = _to_native_out(flat.reshape(Bt * 7, 128), Bt, F, D)
    return jnp.transpose(phys.reshape(F, D, Bt), (2, 0, 1))


# speedup vs baseline: 1.7237x; 1.0491x over previous
import functools

import jax
import jax.numpy as jnp
from jax import lax
from jax.experimental import pallas as pl
from jax.experimental.pallas import tpu as pltpu
from jax.experimental.pallas import tpu_sc as plsc

_NBUF = 2


def _eye128():
    r = lax.broadcasted_iota(jnp.int32, (128, 128), 0)
    c = lax.broadcasted_iota(jnp.int32, (128, 128), 1)
    return jnp.where(r == c, 1.0, 0.0).astype(jnp.float32)


def _mxu_t(x, ident):
    return lax.dot_general(
        ident, x, (((1,), (1,)), ((), ())), preferred_element_type=jnp.float32
    )


def _pack_table(wT, V, D):
    CI = 2048
    NB = (V + CI - 1) // CI

    def body(x_ref, o_ref):
        x = x_ref[...]
        ident = _eye128()
        zs = []
        for q in range(4):
            parts = [
                _mxu_t(x[:, 128 * (4 * q + k) : 128 * (4 * q + k) + 128], ident)
                for k in range(4)
            ]
            zs.append(jnp.concatenate(parts, axis=1))
        o_ref[...] = jnp.concatenate(zs, axis=0)

    return pl.pallas_call(
        body,
        grid=(NB,),
        in_specs=[pl.BlockSpec((D, CI), lambda i: (0, i))],
        out_specs=pl.BlockSpec((512, 4 * D), lambda i: (i, 0)),
        out_shape=jax.ShapeDtypeStruct((NB * 512, 4 * D), jnp.float32),
    )(wT)


def _gather_fn(B, D, VR, CH, num_ch, b_per_w, num_cores):
    mesh = plsc.VectorSubcoreMesh(core_axis_name="c", subcore_axis_name="s")

    @functools.partial(
        pl.kernel,
        mesh=mesh,
        out_type=jax.ShapeDtypeStruct((B, D), jnp.float32),
        scratch_types=[
            pltpu.VMEM((b_per_w,), jnp.int32),
            [pltpu.VMEM((CH, D), jnp.float32) for _ in range(_NBUF)],
            [pltpu.SemaphoreType.DMA for _ in range(_NBUF)],
            [pltpu.SemaphoreType.DMA for _ in range(_NBUF)],
        ],
        compiler_params=pltpu.CompilerParams(use_tc_tiling_on_sc=False),
    )
    def k(idx_hbm, table_hbm, out_hbm, idx_v, rows, g_sems, s_sems):
        wid = lax.axis_index("s") * num_cores + lax.axis_index("c")
        base = wid * b_per_w
        pltpu.sync_copy(idx_hbm.at[pl.ds(base, b_per_w)], idx_v)

        gathers = [None] * num_ch
        stores = [None] * num_ch

        def fire_gather(i):
            s = i % _NBUF
            gathers[i] = pltpu.async_copy(
                table_hbm.at[idx_v.at[pl.ds(i * CH, CH)]], rows[s], g_sems[s]
            )

        for b in range(min(_NBUF, num_ch)):
            fire_gather(b)
        for i in range(num_ch):
            s = i % _NBUF
            gathers[i].wait()
            stores[i] = pltpu.async_copy(
                rows[s], out_hbm.at[pl.ds(base + i * CH, CH)], s_sems[s]
            )
            nxt = i + _NBUF
            if nxt < num_ch:
                stores[i].wait()
                fire_gather(nxt)
        for i in range(max(0, num_ch - _NBUF), num_ch):
            stores[i].wait()

    return k


def _to_native_out(x7, Bt, F, D):
    BB = 128
    FD = F * D
    SL = (7 * 128) // BB if BB != 128 else 7

    def body(x_ref, o_ref):
        x = x_ref[...]
        ident = _eye128()
        y = x.reshape(BB, 7, 128)
        parts = [_mxu_t(y[:, s, :], ident) for s in range(6)]
        parts.append(_mxu_t(y[:, 6, :], ident)[0 : FD - 6 * 128, :])
        o_ref[...] = jnp.concatenate(parts, axis=0)

    return pl.pallas_call(
        body,
        grid=(Bt // BB,),
        in_specs=[pl.BlockSpec((BB * 7, 128), lambda i: (i, 0))],
        out_specs=pl.BlockSpec((FD, BB), lambda i: (0, i)),
        out_shape=jax.ShapeDtypeStruct((FD, Bt), jnp.float32),
    )(x7)


def kernel(indices, weights):
    Bt, F = indices.shape
    V, D = weights.shape
    FP = 28
    B2 = Bt * FP

    t4 = _pack_table(weights.T, V, D)
    VR = t4.shape[0] * 4
    table = t4.reshape(VR, D)

    idx = indices.astype(jnp.int32)
    idx = jnp.pad(idx, ((0, 0), (0, FP - F)), mode="wrap")
    r = idx - (idx % 512) + 4 * (idx % 128) + (idx // 128) % 4
    idx_flat = r.reshape(B2)

    info = plsc.get_sparse_core_info()
    NW = info.num_cores * info.num_subcores
    b_per_w = B2 // NW
    CH = 896
    num_ch = b_per_w // CH
    assert b_per_w % CH == 0 and B2 % NW == 0

    flat = _gather_fn(B2, D, VR, CH, num_ch, b_per_w, info.num_cores)(
        idx_flat, table
    )

    phys = _to_native_out(flat.reshape(Bt * 7, 128), Bt, F, D)
    return jnp.transpose(phys.reshape(F, D, Bt), (2, 0, 1))


# speedup vs baseline: 1.8598x; 1.0790x over previous
import functools

import jax
import jax.numpy as jnp
from jax import lax
from jax.experimental import pallas as pl
from jax.experimental.pallas import tpu as pltpu
from jax.experimental.pallas import tpu_sc as plsc

_NBUF = 2


def _eye128():
    r = lax.broadcasted_iota(jnp.int32, (128, 128), 0)
    c = lax.broadcasted_iota(jnp.int32, (128, 128), 1)
    return jnp.where(r == c, 1.0, 0.0).astype(jnp.float32)


def _mxu_t(x, ident):
    return lax.dot_general(
        ident, x, (((1,), (1,)), ((), ())), preferred_element_type=jnp.float32
    )


def _pack_table(wT, V, D):
    CI = 2048
    NB = (V + CI - 1) // CI

    def body(x_ref, o_ref):
        x = x_ref[...]
        ident = _eye128()
        zs = []
        for q in range(4):
            v = jnp.concatenate(
                [
                    x[:, 128 * (4 * q + k) : 128 * (4 * q + k) + 128]
                    for k in range(4)
                ],
                axis=0,
            )
            zs.append(_mxu_t(v, ident))
        o_ref[...] = jnp.concatenate(zs, axis=0)

    return pl.pallas_call(
        body,
        grid=(NB,),
        in_specs=[pl.BlockSpec((D, CI), lambda i: (0, i))],
        out_specs=pl.BlockSpec((512, 4 * D), lambda i: (i, 0)),
        out_shape=jax.ShapeDtypeStruct((NB * 512, 4 * D), jnp.float32),
    )(wT)


def _gather_fn(B, D, VR, CH, num_ch, b_per_w, num_cores):
    mesh = plsc.VectorSubcoreMesh(core_axis_name="c", subcore_axis_name="s")

    @functools.partial(
        pl.kernel,
        mesh=mesh,
        out_type=jax.ShapeDtypeStruct((B, D), jnp.float32),
        scratch_types=[
            pltpu.VMEM((b_per_w,), jnp.int32),
            [pltpu.VMEM((CH, D), jnp.float32) for _ in range(_NBUF)],
            [pltpu.SemaphoreType.DMA for _ in range(_NBUF)],
            [pltpu.SemaphoreType.DMA for _ in range(_NBUF)],
        ],
        compiler_params=pltpu.CompilerParams(use_tc_tiling_on_sc=False),
    )
    def k(idx_hbm, table_hbm, out_hbm, idx_v, rows, g_sems, s_sems):
        wid = lax.axis_index("s") * num_cores + lax.axis_index("c")
        base = wid * b_per_w
        pltpu.sync_copy(idx_hbm.at[pl.ds(base, b_per_w)], idx_v)

        gathers = [None] * num_ch
        stores = [None] * num_ch

        def fire_gather(i):
            s = i % _NBUF
            gathers[i] = pltpu.async_copy(
                table_hbm.at[idx_v.at[pl.ds(i * CH, CH)]], rows[s], g_sems[s]
            )

        for b in range(min(_NBUF, num_ch)):
            fire_gather(b)
        for i in range(num_ch):
            s = i % _NBUF
            gathers[i].wait()
            stores[i] = pltpu.async_copy(
                rows[s], out_hbm.at[pl.ds(base + i * CH, CH)], s_sems[s]
            )
            nxt = i + _NBUF
            if nxt < num_ch:
                stores[i].wait()
                fire_gather(nxt)
        for i in range(max(0, num_ch - _NBUF), num_ch):
            stores[i].wait()

    return k


def _to_native_out(x7, Bt, F, D):
    BB = 128
    FD = F * D
    SL = (7 * 128) // BB if BB != 128 else 7

    def body(x_ref, o_ref):
        x = x_ref[...]
        ident = _eye128()
        y = x.reshape(BB, 7, 128)
        parts = [_mxu_t(y[:, s, :], ident) for s in range(6)]
        parts.append(_mxu_t(y[:, 6, :], ident)[0 : FD - 6 * 128, :])
        o_ref[...] = jnp.concatenate(parts, axis=0)

    return pl.pallas_call(
        body,
        grid=(Bt // BB,),
        in_specs=[pl.BlockSpec((BB * 7, 128), lambda i: (i, 0))],
        out_specs=pl.BlockSpec((FD, BB), lambda i: (0, i)),
        out_shape=jax.ShapeDtypeStruct((FD, Bt), jnp.float32),
    )(x7)


def kernel(indices, weights):
    Bt, F = indices.shape
    V, D = weights.shape
    FP = 28
    B2 = Bt * FP

    t4 = _pack_table(weights.T, V, D)
    VR = t4.shape[0] * 4
    table = t4.reshape(VR, D)

    idx = indices.astype(jnp.int32)
    idx = jnp.pad(idx, ((0, 0), (0, FP - F)), mode="wrap")
    r = idx - (idx % 512) + 4 * (idx % 128) + (idx // 128) % 4
    idx_flat = r.reshape(B2)

    info = plsc.get_sparse_core_info()
    NW = info.num_cores * info.num_subcores
    b_per_w = B2 // NW
    CH = 896
    num_ch = b_per_w // CH
    assert b_per_w % CH == 0 and B2 % NW == 0

    flat = _gather_fn(B2, D, VR, CH, num_ch, b_per_w, info.num_cores)(
        idx_flat, table
    )

    phys = _to_native_out(flat.reshape(Bt * 7, 128), Bt, F, D)
    return jnp.transpose(phys.reshape(F, D, Bt), (2, 0, 1))


# speedup vs baseline: 3.3752x; 1.8148x over previous
import functools

import jax
import jax.numpy as jnp
from jax import lax
from jax.experimental import pallas as pl
from jax.experimental.pallas import tpu as pltpu
from jax.experimental.pallas import tpu_sc as plsc

_NBUF = 2


def _eye128():
    r = lax.broadcasted_iota(jnp.int32, (128, 128), 0)
    c = lax.broadcasted_iota(jnp.int32, (128, 128), 1)
    return jnp.where(r == c, 1.0, 0.0).astype(jnp.float32)


def _mxu_t(x, ident):
    return lax.dot_general(
        ident, x, (((1,), (1,)), ((), ())), preferred_element_type=jnp.float32
    )


def _pack_table(wT, V, D):
    CI = 8192
    NB = (V + CI - 1) // CI
    NQ = CI // 512

    def body(x_ref, o_ref):
        x = x_ref[...]
        ident = _eye128()
        zs = []
        for q in range(NQ):
            v = jnp.concatenate(
                [
                    x[:, 128 * (4 * q + k) : 128 * (4 * q + k) + 128]
                    for k in range(4)
                ],
                axis=0,
            )
            zs.append(_mxu_t(v, ident))
        o_ref[...] = jnp.concatenate(zs, axis=0)

    return pl.pallas_call(
        body,
        grid=(NB,),
        in_specs=[pl.BlockSpec((D, CI), lambda i: (0, i))],
        out_specs=pl.BlockSpec((CI // 4, 4 * D), lambda i: (i, 0)),
        out_shape=jax.ShapeDtypeStruct((NB * CI // 4, 4 * D), jnp.float32),
    )(wT)


def _gather_fn(B, D, VR, CH, num_ch, b_per_w, num_cores):
    mesh = plsc.VectorSubcoreMesh(core_axis_name="c", subcore_axis_name="s")

    @functools.partial(
        pl.kernel,
        mesh=mesh,
        out_type=jax.ShapeDtypeStruct((B, D), jnp.float32),
        scratch_types=[
            pltpu.VMEM((b_per_w,), jnp.int32),
            [pltpu.VMEM((CH, D), jnp.float32) for _ in range(_NBUF)],
            [pltpu.SemaphoreType.DMA for _ in range(_NBUF)],
            [pltpu.SemaphoreType.DMA for _ in range(_NBUF)],
        ],
        compiler_params=pltpu.CompilerParams(use_tc_tiling_on_sc=False),
    )
    def k(idx_hbm, table_hbm, out_hbm, idx_v, rows, g_sems, s_sems):
        wid = lax.axis_index("s") * num_cores + lax.axis_index("c")
        base = wid * b_per_w
        pltpu.sync_copy(idx_hbm.at[pl.ds(base, b_per_w)], idx_v)

        gathers = [None] * num_ch
        stores = [None] * num_ch

        def fire_gather(i):
            s = i % _NBUF
            gathers[i] = pltpu.async_copy(
                table_hbm.at[idx_v.at[pl.ds(i * CH, CH)]], rows[s], g_sems[s]
            )

        for b in range(min(_NBUF, num_ch)):
            fire_gather(b)
        for i in range(num_ch):
            s = i % _NBUF
            gathers[i].wait()
            stores[i] = pltpu.async_copy(
                rows[s], out_hbm.at[pl.ds(base + i * CH, CH)], s_sems[s]
            )
            nxt = i + _NBUF
            if nxt < num_ch:
                stores[i].wait()
                fire_gather(nxt)
        for i in range(max(0, num_ch - _NBUF), num_ch):
            stores[i].wait()

    return k


def _to_native_out(x7, Bt, F, D):
    BB = 512
    FD = F * D

    def body(x_ref, o_ref):
        x = x_ref[...]
        ident = _eye128()
        y = x.reshape(BB, 7, 128)
        parts = [_mxu_t(y[:, s, :], ident) for s in range(6)]
        parts.append(_mxu_t(y[:, 6, :], ident)[0 : FD - 6 * 128, :])
        o_ref[...] = jnp.concatenate(parts, axis=0)

    return pl.pallas_call(
        body,
        grid=(Bt // BB,),
        in_specs=[pl.BlockSpec((BB * 7, 128), lambda i: (i, 0))],
        out_specs=pl.BlockSpec((FD, BB), lambda i: (0, i)),
        out_shape=jax.ShapeDtypeStruct((FD, Bt), jnp.float32),
    )(x7)


def kernel(indices, weights):
    Bt, F = indices.shape
    V, D = weights.shape
    FP = 28
    B2 = Bt * FP

    t4 = _pack_table(weights.T, V, D)
    VR = t4.shape[0] * 4
    table = t4.reshape(VR, D)

    idx = indices.astype(jnp.int32)
    idx = jnp.pad(idx, ((0, 0), (0, FP - F)), mode="wrap")
    r = idx - (idx % 512) + 4 * (idx % 128) + (idx // 128) % 4
    idx_flat = r.reshape(B2)

    info = plsc.get_sparse_core_info()
    NW = info.num_cores * info.num_subcores
    b_per_w = B2 // NW
    CH = 896
    num_ch = b_per_w // CH
    assert b_per_w % CH == 0 and B2 % NW == 0

    flat = _gather_fn(B2, D, VR, CH, num_ch, b_per_w, info.num_cores)(
        idx_flat, table
    )

    phys = _to_native_out(flat.reshape(Bt * 7, 128), Bt, F, D)
    return jnp.transpose(phys.reshape(F, D, Bt), (2, 0, 1))


# speedup vs baseline: 4.3281x; 1.2824x over previous
import functools

import jax
import jax.numpy as jnp
from jax import lax
from jax.experimental import pallas as pl
from jax.experimental.pallas import tpu as pltpu
from jax.experimental.pallas import tpu_sc as plsc

_NBUF = 2


def _eye128():
    r = lax.broadcasted_iota(jnp.int32, (128, 128), 0)
    c = lax.broadcasted_iota(jnp.int32, (128, 128), 1)
    return jnp.where(r == c, 1.0, 0.0).astype(jnp.float32)


def _mxu_t(x, ident):
    return lax.dot_general(
        ident, x, (((1,), (1,)), ((), ())), preferred_element_type=jnp.float32
    )


def _pack_table(wT, V, D):
    CI = 32768
    NB = (V + CI - 1) // CI
    NQ = CI // 512

    def body(x_ref, o_ref):
        x = x_ref[...]
        ident = _eye128()
        zs = []
        for q in range(NQ):
            v = jnp.concatenate(
                [
                    x[:, 128 * (4 * q + k) : 128 * (4 * q + k) + 128]
                    for k in range(4)
                ],
                axis=0,
            )
            zs.append(_mxu_t(v, ident))
        o_ref[...] = jnp.concatenate(zs, axis=0)

    return pl.pallas_call(
        body,
        grid=(NB,),
        in_specs=[pl.BlockSpec((D, CI), lambda i: (0, i))],
        out_specs=pl.BlockSpec((CI // 4, 4 * D), lambda i: (i, 0)),
        out_shape=jax.ShapeDtypeStruct((NB * CI // 4, 4 * D), jnp.float32),
    )(wT)


def _gather_fn(B, D, VR, CH, num_ch, b_per_w, num_cores):
    mesh = plsc.VectorSubcoreMesh(core_axis_name="c", subcore_axis_name="s")

    @functools.partial(
        pl.kernel,
        mesh=mesh,
        out_type=jax.ShapeDtypeStruct((B, D), jnp.float32),
        scratch_types=[
            pltpu.VMEM((b_per_w,), jnp.int32),
            [pltpu.VMEM((CH, D), jnp.float32) for _ in range(_NBUF)],
            [pltpu.SemaphoreType.DMA for _ in range(_NBUF)],
            [pltpu.SemaphoreType.DMA for _ in range(_NBUF)],
        ],
        compiler_params=pltpu.CompilerParams(use_tc_tiling_on_sc=False),
    )
    def k(idx_hbm, table_hbm, out_hbm, idx_v, rows, g_sems, s_sems):
        wid = lax.axis_index("s") * num_cores + lax.axis_index("c")
        base = wid * b_per_w
        pltpu.sync_copy(idx_hbm.at[pl.ds(base, b_per_w)], idx_v)

        gathers = [None] * num_ch
        stores = [None] * num_ch

        def fire_gather(i):
            s = i % _NBUF
            gathers[i] = pltpu.async_copy(
                table_hbm.at[idx_v.at[pl.ds(i * CH, CH)]], rows[s], g_sems[s]
            )

        for b in range(min(_NBUF, num_ch)):
            fire_gather(b)
        for i in range(num_ch):
            s = i % _NBUF
            gathers[i].wait()
            stores[i] = pltpu.async_copy(
                rows[s], out_hbm.at[pl.ds(base + i * CH, CH)], s_sems[s]
            )
            nxt = i + _NBUF
            if nxt < num_ch:
                stores[i].wait()
                fire_gather(nxt)
        for i in range(max(0, num_ch - _NBUF), num_ch):
            stores[i].wait()

    return k


def _to_native_out(x7, Bt, F, D):
    BB = 2048
    FD = F * D

    def body(x_ref, o_ref):
        x = x_ref[...]
        ident = _eye128()
        y = x.reshape(BB, 7, 128)
        parts = [_mxu_t(y[:, s, :], ident) for s in range(6)]
        parts.append(_mxu_t(y[:, 6, :], ident)[0 : FD - 6 * 128, :])
        o_ref[...] = jnp.concatenate(parts, axis=0)

    return pl.pallas_call(
        body,
        grid=(Bt // BB,),
        in_specs=[pl.BlockSpec((BB * 7, 128), lambda i: (i, 0))],
        out_specs=pl.BlockSpec((FD, BB), lambda i: (0, i)),
        out_shape=jax.ShapeDtypeStruct((FD, Bt), jnp.float32),
    )(x7)


def kernel(indices, weights):
    Bt, F = indices.shape
    V, D = weights.shape
    FP = 28
    B2 = Bt * FP

    t4 = _pack_table(weights.T, V, D)
    VR = t4.shape[0] * 4
    table = t4.reshape(VR, D)

    idx = indices.astype(jnp.int32)
    idx = jnp.pad(idx, ((0, 0), (0, FP - F)), mode="wrap")
    r = idx - (idx % 512) + 4 * (idx % 128) + (idx // 128) % 4
    idx_flat = r.reshape(B2)

    info = plsc.get_sparse_core_info()
    NW = info.num_cores * info.num_subcores
    b_per_w = B2 // NW
    CH = 896
    num_ch = b_per_w // CH
    assert b_per_w % CH == 0 and B2 % NW == 0

    flat = _gather_fn(B2, D, VR, CH, num_ch, b_per_w, info.num_cores)(
        idx_flat, table
    )

    phys = _to_native_out(flat.reshape(Bt * 7, 128), Bt, F, D)
    return jnp.transpose(phys.reshape(F, D, Bt), (2, 0, 1))


# speedup vs baseline: 4.3362x; 1.0019x over previous
import functools

import jax
import jax.numpy as jnp
from jax import lax
from jax.experimental import pallas as pl
from jax.experimental.pallas import tpu as pltpu
from jax.experimental.pallas import tpu_sc as plsc

_NBUF = 2


def _eye128():
    r = lax.broadcasted_iota(jnp.int32, (128, 128), 0)
    c = lax.broadcasted_iota(jnp.int32, (128, 128), 1)
    return jnp.where(r == c, 1.0, 0.0).astype(jnp.float32)


def _mxu_t(x, ident):
    return lax.dot_general(
        ident, x, (((1,), (1,)), ((), ())), preferred_element_type=jnp.float32
    )


def _pack_table(wT, V, D):
    CI = 32768
    NB = (V + CI - 1) // CI
    NQ = CI // 512

    def body(x_ref, o_ref):
        x = x_ref[...]
        ident = _eye128()
        zs = []
        for q in range(NQ):
            v = jnp.concatenate(
                [
                    x[:, 128 * (4 * q + k) : 128 * (4 * q + k) + 128]
                    for k in range(4)
                ],
                axis=0,
            )
            zs.append(_mxu_t(v, ident))
        o_ref[...] = jnp.concatenate(zs, axis=0)

    return pl.pallas_call(
        body,
        grid=(NB,),
        in_specs=[pl.BlockSpec((D, CI), lambda i: (0, i))],
        out_specs=pl.BlockSpec((CI // 4, 4 * D), lambda i: (i, 0)),
        out_shape=jax.ShapeDtypeStruct((NB * CI // 4, 4 * D), jnp.float32),
    )(wT)


def _gather_fn(B, D, VR, CH, num_ch, b_per_w, num_cores):
    mesh = plsc.VectorSubcoreMesh(core_axis_name="c", subcore_axis_name="s")

    @functools.partial(
        pl.kernel,
        mesh=mesh,
        out_type=jax.ShapeDtypeStruct((B, D), jnp.float32),
        scratch_types=[
            pltpu.VMEM((b_per_w,), jnp.int32),
            [pltpu.VMEM((CH, D), jnp.float32) for _ in range(_NBUF)],
            [pltpu.SemaphoreType.DMA for _ in range(_NBUF)],
            [pltpu.SemaphoreType.DMA for _ in range(_NBUF)],
        ],
        compiler_params=pltpu.CompilerParams(use_tc_tiling_on_sc=False),
    )
    def k(idx_hbm, table_hbm, out_hbm, idx_v, rows, g_sems, s_sems):
        wid = lax.axis_index("s") * num_cores + lax.axis_index("c")
        base = wid * b_per_w
        pltpu.sync_copy(idx_hbm.at[pl.ds(base, b_per_w)], idx_v)

        gathers = [None] * num_ch
        stores = [None] * num_ch

        def fire_gather(i):
            s = i % _NBUF
            gathers[i] = pltpu.async_copy(
                table_hbm.at[idx_v.at[pl.ds(i * CH, CH)]], rows[s], g_sems[s]
            )

        for b in range(min(_NBUF, num_ch)):
            fire_gather(b)
        for i in range(num_ch):
            s = i % _NBUF
            gathers[i].wait()
            stores[i] = pltpu.async_copy(
                rows[s], out_hbm.at[pl.ds(base + i * CH, CH)], s_sems[s]
            )
            nxt = i + _NBUF
            if nxt < num_ch:
                stores[i].wait()
                fire_gather(nxt)
        for i in range(max(0, num_ch - _NBUF), num_ch):
            stores[i].wait()

    return k


def _to_native_out(x7, Bt, F, D):
    BB = 2048
    FD = F * D

    def body(x_ref, o_ref):
        x = x_ref[...]
        ident = _eye128()
        y = x.reshape(BB, 7, 128)
        parts = [_mxu_t(y[:, s, :], ident) for s in range(6)]
        parts.append(_mxu_t(y[:, 6, :], ident)[0 : FD - 6 * 128, :])
        o_ref[...] = jnp.concatenate(parts, axis=0)

    return pl.pallas_call(
        body,
        grid=(Bt // BB,),
        in_specs=[pl.BlockSpec((BB * 7, 128), lambda i: (i, 0))],
        out_specs=pl.BlockSpec((FD, BB), lambda i: (0, i)),
        out_shape=jax.ShapeDtypeStruct((FD, Bt), jnp.float32),
    )(x7)


def kernel(indices, weights):
    Bt, F = indices.shape
    V, D = weights.shape
    FP = 28
    B2 = Bt * FP

    t4 = _pack_table(weights.T, V, D)
    VR = t4.shape[0] * 4
    table = t4.reshape(VR, D)

    idx = indices.astype(jnp.int32)
    idx = jnp.pad(idx, ((0, 0), (0, FP - F)), mode="wrap")
    r = idx - (idx % 512) + 4 * (idx % 128) + (idx // 128) % 4
    idx_flat = r.reshape(B2)

    info = plsc.get_sparse_core_info()
    NW = info.num_cores * info.num_subcores
    b_per_w = B2 // NW
    CH = 1792
    num_ch = b_per_w // CH
    assert b_per_w % CH == 0 and B2 % NW == 0

    flat = _gather_fn(B2, D, VR, CH, num_ch, b_per_w, info.num_cores)(
        idx_flat, table
    )

    phys = _to_native_out(flat.reshape(Bt * 7, 128), Bt, F, D)
    return jnp.transpose(phys.reshape(F, D, Bt), (2, 0, 1))


# speedup vs baseline: 4.4216x; 1.0197x over previous
import functools

import jax
import jax.numpy as jnp
from jax import lax
from jax.experimental import pallas as pl
from jax.experimental.pallas import tpu as pltpu
from jax.experimental.pallas import tpu_sc as plsc

_NBUF = 2


def _eye128():
    r = lax.broadcasted_iota(jnp.int32, (128, 128), 0)
    c = lax.broadcasted_iota(jnp.int32, (128, 128), 1)
    return jnp.where(r == c, 1.0, 0.0).astype(jnp.float32)


def _mxu_t(x, ident):
    return lax.dot_general(
        ident, x, (((1,), (1,)), ((), ())), preferred_element_type=jnp.float32
    )


def _pack_table(wT, V, D):
    CI = 32768
    NB = (V + CI - 1) // CI
    NQ = CI // 512

    def body(x_ref, o_ref):
        x = x_ref[...]
        ident = _eye128()
        zs = []
        for q in range(NQ):
            v = jnp.concatenate(
                [
                    x[:, 128 * (4 * q + k) : 128 * (4 * q + k) + 128]
                    for k in range(4)
                ],
                axis=0,
            )
            zs.append(_mxu_t(v, ident) if q % 2 == 0 else v.T)
        o_ref[...] = jnp.concatenate(zs, axis=0)

    return pl.pallas_call(
        body,
        grid=(NB,),
        in_specs=[pl.BlockSpec((D, CI), lambda i: (0, i))],
        out_specs=pl.BlockSpec((CI // 4, 4 * D), lambda i: (i, 0)),
        out_shape=jax.ShapeDtypeStruct((NB * CI // 4, 4 * D), jnp.float32),
    )(wT)


def _gather_fn(B, D, VR, CH, num_ch, b_per_w, num_cores):
    mesh = plsc.VectorSubcoreMesh(core_axis_name="c", subcore_axis_name="s")

    @functools.partial(
        pl.kernel,
        mesh=mesh,
        out_type=jax.ShapeDtypeStruct((B, D), jnp.float32),
        scratch_types=[
            pltpu.VMEM((b_per_w,), jnp.int32),
            [pltpu.VMEM((CH, D), jnp.float32) for _ in range(_NBUF)],
            [pltpu.SemaphoreType.DMA for _ in range(_NBUF)],
            [pltpu.SemaphoreType.DMA for _ in range(_NBUF)],
        ],
        compiler_params=pltpu.CompilerParams(use_tc_tiling_on_sc=False),
    )
    def k(idx_hbm, table_hbm, out_hbm, idx_v, rows, g_sems, s_sems):
        wid = lax.axis_index("s") * num_cores + lax.axis_index("c")
        base = wid * b_per_w
        pltpu.sync_copy(idx_hbm.at[pl.ds(base, b_per_w)], idx_v)

        gathers = [None] * num_ch
        stores = [None] * num_ch

        def fire_gather(i):
            s = i % _NBUF
            gathers[i] = pltpu.async_copy(
                table_hbm.at[idx_v.at[pl.ds(i * CH, CH)]], rows[s], g_sems[s]
            )

        for b in range(min(_NBUF, num_ch)):
            fire_gather(b)
        for i in range(num_ch):
            s = i % _NBUF
            gathers[i].wait()
            stores[i] = pltpu.async_copy(
                rows[s], out_hbm.at[pl.ds(base + i * CH, CH)], s_sems[s]
            )
            nxt = i + _NBUF
            if nxt < num_ch:
                stores[i].wait()
                fire_gather(nxt)
        for i in range(max(0, num_ch - _NBUF), num_ch):
            stores[i].wait()

    return k


def _to_native_out(x7, Bt, F, D):
    BB = 2048
    FD = F * D

    def body(x_ref, o_ref):
        x = x_ref[...]
        ident = _eye128()
        y = x.reshape(BB, 7, 128)
        parts = [_mxu_t(y[:, s, :], ident) for s in range(6)]
        parts.append(_mxu_t(y[:, 6, :], ident)[0 : FD - 6 * 128, :])
        o_ref[...] = jnp.concatenate(parts, axis=0)

    return pl.pallas_call(
        body,
        grid=(Bt // BB,),
        in_specs=[pl.BlockSpec((BB * 7, 128), lambda i: (i, 0))],
        out_specs=pl.BlockSpec((FD, BB), lambda i: (0, i)),
        out_shape=jax.ShapeDtypeStruct((FD, Bt), jnp.float32),
    )(x7)


def kernel(indices, weights):
    Bt, F = indices.shape
    V, D = weights.shape
    FP = 28
    B2 = Bt * FP

    t4 = _pack_table(weights.T, V, D)
    VR = t4.shape[0] * 4
    table = t4.reshape(VR, D)

    idx = indices.astype(jnp.int32)
    idx = jnp.pad(idx, ((0, 0), (0, FP - F)), mode="wrap")
    r = idx - (idx % 512) + 4 * (idx % 128) + (idx // 128) % 4
    idx_flat = r.reshape(B2)

    info = plsc.get_sparse_core_info()
    NW = info.num_cores * info.num_subcores
    b_per_w = B2 // NW
    CH = 1792
    num_ch = b_per_w // CH
    assert b_per_w % CH == 0 and B2 % NW == 0

    flat = _gather_fn(B2, D, VR, CH, num_ch, b_per_w, info.num_cores)(
        idx_flat, table
    )

    phys = _to_native_out(flat.reshape(Bt * 7, 128), Bt, F, D)
    return jnp.transpose(phys.reshape(F, D, Bt), (2, 0, 1))
